# baseline (speedup 1.0000x reference)
import functools

import jax
import jax.numpy as jnp
from jax import lax
from jax.experimental import pallas as pl
from jax.experimental.pallas import tpu as pltpu
from jax.experimental.pallas import tpu_sc as plsc

KEEPK = 64
BB, TT, CC = 4, 2048, 2048
HB = 2
TBLK = 1024
RCH = 256


def _attn_sel_body(b0, x_ref, w_ref, b_ref, s_ref, idxg_ref):
    bi = pl.program_id(0) + b0
    xb = x_ref[0]
    a_col = lax.dot_general(
        xb, w_ref[...], (((1,), (0,)), ((), ())),
        preferred_element_type=jnp.float32) + b_ref[0, 0]
    a_row = lax.transpose(a_col, (1, 0))
    i_row = lax.broadcasted_iota(jnp.int32, (1, TT), 1)

    acc = jnp.zeros((RCH, TT), jnp.int32)
    for k in range(TT // RCH):
        ac = lax.slice(a_col, (k * RCH, 0), ((k + 1) * RCH, 1))
        ic = lax.broadcasted_iota(jnp.int32, (RCH, 1), 0) + k * RCH
        beats = (ac > a_row) | ((ac == a_row) & (ic < i_row))
        acc = acc + beats.astype(jnp.int32)
    rank = jnp.sum(acc, axis=0, keepdims=True)
    sel = rank < KEEPK
    m = sel.astype(jnp.int32)

    cum = m
    sh = 1
    while sh < TT:
        cum = cum + jnp.concatenate(
            [jnp.zeros((1, sh), jnp.int32), cum[:, :TT - sh]], axis=1)
        sh *= 2
    pos = cum - m

    sub_k = lax.broadcasted_iota(jnp.int32, (KEEPK, TT), 0)
    onehot = (pos == sub_k) & sel
    s_ref[0] = onehot.astype(jnp.bfloat16)
    gidx = jnp.where(onehot, i_row + bi * TT, 0)
    idxg_ref[0] = jnp.sum(gidx, axis=1, keepdims=True)


def _attn_sel(xh, w, b2, b0):
    return pl.pallas_call(
        functools.partial(_attn_sel_body, b0),
        grid=(HB,),
        in_specs=[
            pl.BlockSpec((1, TT, CC), lambda b: (b, 0, 0)),
            pl.BlockSpec((CC, 1), lambda b: (0, 0)),
            pl.BlockSpec((1, 1), lambda b: (0, 0)),
        ],
        out_specs=[
            pl.BlockSpec((1, KEEPK, CC), lambda b: (b, 0, 0)),
            pl.BlockSpec((1, KEEPK, 1), lambda b: (b, 0, 0)),
        ],
        out_shape=[
            jax.ShapeDtypeStruct((HB, KEEPK, CC), jnp.bfloat16),
            jax.ShapeDtypeStruct((HB, KEEPK, 1), jnp.int32),
        ],
    )(xh, w, b2)


def _sc_gather(x2d, idx_flat):
    info = plsc.get_sparse_core_info()
    nrows = HB * KEEPK
    bpw = 8
    nactive = nrows // bpw
    mesh = plsc.VectorSubcoreMesh(core_axis_name="c", subcore_axis_name="s")

    @functools.partial(
        pl.kernel,
        mesh=mesh,
        out_type=jax.ShapeDtypeStruct((nrows, CC), jnp.float32),
        scratch_types=[
            pltpu.VMEM((bpw,), jnp.int32),
            pltpu.VMEM((bpw, CC), jnp.float32),
            pltpu.SemaphoreType.DMA,
        ],
    )
    def gk(x_hbm, idx_hbm, out_hbm, idx_v, rows_v, sem):
        wid = lax.axis_index("s") * info.num_cores + lax.axis_index("c")

        @pl.when(wid < nactive)
        def _():
            base = wid * bpw
            pltpu.sync_copy(idx_hbm.at[pl.ds(base, bpw)], idx_v)
            pltpu.async_copy(x_hbm.at[idx_v], rows_v, sem).wait()
            pltpu.sync_copy(rows_v, out_hbm.at[pl.ds(base, bpw)])

    return gk(x2d, idx_flat)


def _scatter_body(g_ref, s_ref, *rest):
    o_ref = rest[-1]
    onehot = s_ref[0]
    g = g_ref[...]
    hi32 = lax.bitcast_convert_type(
        lax.bitcast_convert_type(g, jnp.uint32) & jnp.uint32(0xFFFF0000),
        jnp.float32)
    hi = hi32.astype(jnp.bfloat16)
    lo = (g - hi32).astype(jnp.bfloat16)
    ghl = jnp.concatenate([hi, lo], axis=0)
    ohh = jnp.concatenate([onehot, onehot], axis=0)
    o_ref[0] = lax.dot_general(
        ghl, ohh, (((0,), (0,)), ((), ())),
        preferred_element_type=jnp.float32)


def _scatter_first(g, s):
    return pl.pallas_call(
        _scatter_body,
        grid=(HB, TT // TBLK),
        in_specs=[
            pl.BlockSpec((KEEPK, TBLK), lambda b, t: (b, t)),
            pl.BlockSpec((1, KEEPK, CC), lambda b, t: (b, 0, 0)),
        ],
        out_specs=pl.BlockSpec((1, TBLK, CC), lambda b, t: (b, t, 0)),
        out_shape=jax.ShapeDtypeStruct((BB, TT, CC), jnp.float32),
    )(g, s)


def _scatter_second(g, s, prev):
    return pl.pallas_call(
        _scatter_body,
        grid=(HB, TT // TBLK),
        in_specs=[
            pl.BlockSpec((KEEPK, TBLK), lambda b, t: (b, t)),
            pl.BlockSpec((1, KEEPK, CC), lambda b, t: (b, 0, 0)),
            pl.BlockSpec(memory_space=pl.ANY),
        ],
        out_specs=pl.BlockSpec((1, TBLK, CC), lambda b, t: (b + HB, t, 0)),
        out_shape=jax.ShapeDtypeStruct((BB, TT, CC), jnp.float32),
        input_output_aliases={2: 0},
    )(g, s, prev)


def kernel(x, W, b):
    b2 = b.reshape(1, 1)
    x2d = x.reshape(BB * TT, CC)
    s01, i01 = _attn_sel(x[0:HB], W, b2, 0)
    s23, i23 = _attn_sel(x[HB:BB], W, b2, HB)
    g01 = _sc_gather(x2d, i01.reshape(HB * KEEPK))
    g23 = _sc_gather(x2d, i23.reshape(HB * KEEPK))
    o1 = _scatter_first(g01, s01)
    return _scatter_second(g23, s23, o1)


# speedup vs baseline: 1.5007x; 1.5007x over previous
import functools

import jax
import jax.numpy as jnp
from jax import lax
from jax.experimental import pallas as pl
from jax.experimental.pallas import tpu as pltpu
from jax.experimental.pallas import tpu_sc as plsc

KEEPK = 64
BB, TT, CC = 4, 2048, 2048
HB = 2
TBLK = 1024
RCH = 256


def _attn_sel_body(b0, x_ref, w_ref, b_ref, s_ref, idxg_ref):
    bi = pl.program_id(0) + b0
    xb = x_ref[0]
    a_col = lax.dot_general(
        xb, w_ref[...], (((1,), (0,)), ((), ())),
        preferred_element_type=jnp.float32) + b_ref[0, 0]
    a_row = lax.transpose(a_col, (1, 0))
    i_row = lax.broadcasted_iota(jnp.int32, (1, TT), 1)

    acc = jnp.zeros((RCH, TT), jnp.int32)
    for k in range(TT // RCH):
        ac = lax.slice(a_col, (k * RCH, 0), ((k + 1) * RCH, 1))
        ic = lax.broadcasted_iota(jnp.int32, (RCH, 1), 0) + k * RCH
        beats = (ac > a_row) | ((ac == a_row) & (ic < i_row))
        acc = acc + beats.astype(jnp.int32)
    rank = jnp.sum(acc, axis=0, keepdims=True)
    sel = rank < KEEPK
    m = sel.astype(jnp.int32)

    cum = m
    sh = 1
    while sh < TT:
        cum = cum + jnp.concatenate(
            [jnp.zeros((1, sh), jnp.int32), cum[:, :TT - sh]], axis=1)
        sh *= 2
    pos = cum - m

    sub_k = lax.broadcasted_iota(jnp.int32, (KEEPK, TT), 0)
    onehot = (pos == sub_k) & sel
    s_ref[0] = onehot.astype(jnp.bfloat16)
    gidx = jnp.where(onehot, i_row + bi * TT, 0)
    idxg_ref[0] = jnp.sum(gidx, axis=1, keepdims=True)


def _attn_sel(xh, w, b2, b0):
    return pl.pallas_call(
        functools.partial(_attn_sel_body, b0),
        grid=(HB,),
        in_specs=[
            pl.BlockSpec((1, TT, CC), lambda b: (b + b0, 0, 0)),
            pl.BlockSpec((CC, 1), lambda b: (0, 0)),
            pl.BlockSpec((1, 1), lambda b: (0, 0)),
        ],
        out_specs=[
            pl.BlockSpec((1, KEEPK, CC), lambda b: (b, 0, 0)),
            pl.BlockSpec((1, KEEPK, 1), lambda b: (b, 0, 0)),
        ],
        out_shape=[
            jax.ShapeDtypeStruct((HB, KEEPK, CC), jnp.bfloat16),
            jax.ShapeDtypeStruct((HB, KEEPK, 1), jnp.int32),
        ],
    )(xh, w, b2)


def _sc_gather(x2d, idx_flat):
    info = plsc.get_sparse_core_info()
    nrows = HB * KEEPK
    bpw = 8
    nactive = nrows // bpw
    mesh = plsc.VectorSubcoreMesh(core_axis_name="c", subcore_axis_name="s")

    @functools.partial(
        pl.kernel,
        mesh=mesh,
        out_type=jax.ShapeDtypeStruct((nrows, CC), jnp.float32),
        scratch_types=[
            pltpu.VMEM((bpw,), jnp.int32),
            pltpu.VMEM((bpw, CC), jnp.float32),
            pltpu.SemaphoreType.DMA,
        ],
    )
    def gk(x_hbm, idx_hbm, out_hbm, idx_v, rows_v, sem):
        wid = lax.axis_index("s") * info.num_cores + lax.axis_index("c")

        @pl.when(wid < nactive)
        def _():
            base = wid * bpw
            pltpu.sync_copy(idx_hbm.at[pl.ds(base, bpw)], idx_v)
            pltpu.async_copy(x_hbm.at[idx_v], rows_v, sem).wait()
            pltpu.sync_copy(rows_v, out_hbm.at[pl.ds(base, bpw)])

    return gk(x2d, idx_flat)


def _scatter_body(g_ref, s_ref, *rest):
    o_ref = rest[-1]
    onehot = s_ref[0]
    g = g_ref[...]
    hi32 = lax.bitcast_convert_type(
        lax.bitcast_convert_type(g, jnp.uint32) & jnp.uint32(0xFFFF0000),
        jnp.float32)
    hi = hi32.astype(jnp.bfloat16)
    lo = (g - hi32).astype(jnp.bfloat16)
    ghl = jnp.concatenate([hi, lo], axis=0)
    ohh = jnp.concatenate([onehot, onehot], axis=0)
    o_ref[0] = lax.dot_general(
        ghl, ohh, (((0,), (0,)), ((), ())),
        preferred_element_type=jnp.float32)


def _scatter_first(g, s):
    return pl.pallas_call(
        _scatter_body,
        grid=(HB, TT // TBLK),
        in_specs=[
            pl.BlockSpec((KEEPK, TBLK), lambda b, t: (b, t)),
            pl.BlockSpec((1, KEEPK, CC), lambda b, t: (b, 0, 0)),
        ],
        out_specs=pl.BlockSpec((1, TBLK, CC), lambda b, t: (b, t, 0)),
        out_shape=jax.ShapeDtypeStruct((BB, TT, CC), jnp.float32),
    )(g, s)


def _scatter_second(g, s, prev):
    return pl.pallas_call(
        _scatter_body,
        grid=(HB, TT // TBLK),
        in_specs=[
            pl.BlockSpec((KEEPK, TBLK), lambda b, t: (b, t)),
            pl.BlockSpec((1, KEEPK, CC), lambda b, t: (b, 0, 0)),
            pl.BlockSpec(memory_space=pl.ANY),
        ],
        out_specs=pl.BlockSpec((1, TBLK, CC), lambda b, t: (b + HB, t, 0)),
        out_shape=jax.ShapeDtypeStruct((BB, TT, CC), jnp.float32),
        input_output_aliases={2: 0},
    )(g, s, prev)


def kernel(x, W, b):
    b2 = b.reshape(1, 1)
    x2d = x.reshape(BB * TT, CC)
    s01, i01 = _attn_sel(x, W, b2, 0)
    s23, i23 = _attn_sel(x, W, b2, HB)
    g01 = _sc_gather(x2d, i01.reshape(HB * KEEPK))
    g23 = _sc_gather(x2d, i23.reshape(HB * KEEPK))
    o1 = _scatter_first(g01, s01)
    return _scatter_second(g23, s23, o1)


# speedup vs baseline: 1.6593x; 1.1057x over previous
import functools

import jax
import jax.numpy as jnp
from jax import lax
from jax.experimental import pallas as pl
from jax.experimental.pallas import tpu as pltpu
from jax.experimental.pallas import tpu_sc as plsc

KEEPK = 64
BB, TT, CC = 4, 2048, 2048
TBLK = 1024
RCH = 256


def _attn_sel_body(x_ref, w_ref, b_ref, s_ref, idxg_ref):
    bi = pl.program_id(0)
    xb = x_ref[0]
    a_col = lax.dot_general(
        xb, w_ref[...], (((1,), (0,)), ((), ())),
        preferred_element_type=jnp.float32) + b_ref[0, 0]
    a_row = lax.transpose(a_col, (1, 0))
    i_row = lax.broadcasted_iota(jnp.int32, (1, TT), 1)

    acc = jnp.zeros((RCH, TT), jnp.int32)
    for k in range(TT // RCH):
        ac = lax.slice(a_col, (k * RCH, 0), ((k + 1) * RCH, 1))
        ic = lax.broadcasted_iota(jnp.int32, (RCH, 1), 0) + k * RCH
        beats = (ac > a_row) | ((ac == a_row) & (ic < i_row))
        acc = acc + beats.astype(jnp.int32)
    rank = jnp.sum(acc, axis=0, keepdims=True)
    sel = rank < KEEPK
    m = sel.astype(jnp.int32)

    cum = m
    sh = 1
    while sh < TT:
        cum = cum + jnp.concatenate(
            [jnp.zeros((1, sh), jnp.int32), cum[:, :TT - sh]], axis=1)
        sh *= 2
    pos = cum - m

    sub_k = lax.broadcasted_iota(jnp.int32, (KEEPK, TT), 0)
    onehot = (pos == sub_k) & sel
    s_ref[0] = onehot.astype(jnp.bfloat16)
    gidx = jnp.where(onehot, i_row + bi * TT, 0)
    idxg_ref[0] = jnp.sum(gidx, axis=1, keepdims=True)


def _attn_sel(x, w, b2):
    return pl.pallas_call(
        _attn_sel_body,
        grid=(BB,),
        in_specs=[
            pl.BlockSpec((1, TT, CC), lambda b: (b, 0, 0)),
            pl.BlockSpec((CC, 1), lambda b: (0, 0)),
            pl.BlockSpec((1, 1), lambda b: (0, 0)),
        ],
        out_specs=[
            pl.BlockSpec((1, KEEPK, CC), lambda b: (b, 0, 0)),
            pl.BlockSpec((1, KEEPK, 1), lambda b: (b, 0, 0)),
        ],
        out_shape=[
            jax.ShapeDtypeStruct((BB, KEEPK, CC), jnp.bfloat16),
            jax.ShapeDtypeStruct((BB, KEEPK, 1), jnp.int32),
        ],
    )(x, w, b2)


def _sc_gather(x2d, idx_flat):
    info = plsc.get_sparse_core_info()
    nw = info.num_cores * info.num_subcores
    nrows = BB * KEEPK
    bpw = nrows // nw
    mesh = plsc.VectorSubcoreMesh(core_axis_name="c", subcore_axis_name="s")

    @functools.partial(
        pl.kernel,
        mesh=mesh,
        out_type=jax.ShapeDtypeStruct((nrows, CC), jnp.float32),
        scratch_types=[
            pltpu.VMEM((bpw,), jnp.int32),
            pltpu.VMEM((bpw, CC), jnp.float32),
            pltpu.SemaphoreType.DMA,
        ],
    )
    def gk(x_hbm, idx_hbm, out_hbm, idx_v, rows_v, sem):
        wid = lax.axis_index("s") * info.num_cores + lax.axis_index("c")
        base = wid * bpw
        pltpu.sync_copy(idx_hbm.at[pl.ds(base, bpw)], idx_v)
        pltpu.async_copy(x_hbm.at[idx_v], rows_v, sem).wait()
        pltpu.sync_copy(rows_v, out_hbm.at[pl.ds(base, bpw)])

    return gk(x2d, idx_flat)


def _scatter_body(g_ref, s_ref, o_ref):
    onehot = s_ref[0]
    g = g_ref[...]
    hi32 = lax.bitcast_convert_type(
        lax.bitcast_convert_type(g, jnp.uint32) & jnp.uint32(0xFFFF0000),
        jnp.float32)
    hi = hi32.astype(jnp.bfloat16)
    lo = (g - hi32).astype(jnp.bfloat16)
    ghl = jnp.concatenate([hi, lo], axis=0)
    ohh = jnp.concatenate([onehot, onehot], axis=0)
    o_ref[0] = lax.dot_general(
        ghl, ohh, (((0,), (0,)), ((), ())),
        preferred_element_type=jnp.float32)


def _scatter(g, s):
    return pl.pallas_call(
        _scatter_body,
        grid=(BB, TT // TBLK),
        in_specs=[
            pl.BlockSpec((KEEPK, TBLK), lambda b, t: (b, t)),
            pl.BlockSpec((1, KEEPK, CC), lambda b, t: (b, 0, 0)),
        ],
        out_specs=pl.BlockSpec((1, TBLK, CC), lambda b, t: (b, t, 0)),
        out_shape=jax.ShapeDtypeStruct((BB, TT, CC), jnp.float32),
    )(g, s)


def kernel(x, W, b):
    s, idx_g = _attn_sel(x, W, b.reshape(1, 1))
    g = _sc_gather(x.reshape(BB * TT, CC), idx_g.reshape(BB * KEEPK))
    return _scatter(g, s)
